# baseline (speedup 1.0000x reference)
import jax
import jax.numpy as jnp
from jax.experimental import pallas as pl
from jax.experimental.pallas import tpu as pltpu

T_B = 0.9
_EPS_ADJ = 1e-6
_EPS_DEN = 1e-6
_N = 900
_NP = 1024
_CHUNK = 128
_NCH = _NP // _CHUNK


def _body(bb8_ref, crows_ref, adj_ref, agg_ref, ab_s):
    s0 = jnp.float32(0.0)
    for ci in range(_NCH):
        r0 = ci * _CHUNK
        c = bb8_ref[0, r0:r0 + _CHUNK, :]
        cxi = c[:, 0:1]
        cyi = c[:, 1:2]
        wi = c[:, 2:3]
        hi = c[:, 3:4]
        x1i = cxi - 0.5 * wi
        y1i = cyi - 0.5 * hi
        x2i = cxi + 0.5 * wi
        y2i = cyi + 0.5 * hi
        area_i = (x2i - x1i) * (y2i - y1i)

        def giou_tile(cj, s, r0=r0, x1i=x1i, y1i=y1i, x2i=x2i, y2i=y2i,
                      area_i=area_i, ci=ci):
            c0 = cj * _CHUNK
            jc = crows_ref[0, 0:4, pl.ds(c0, _CHUNK)]
            x1j = jc[0:1, :] - 0.5 * jc[2:3, :]
            y1j = jc[1:2, :] - 0.5 * jc[3:4, :]
            x2j = jc[0:1, :] + 0.5 * jc[2:3, :]
            y2j = jc[1:2, :] + 0.5 * jc[3:4, :]
            area_j = (x2j - x1j) * (y2j - y1j)
            wx = jnp.maximum(jnp.minimum(x2i, x2j) - jnp.maximum(x1i, x1j), 0.0)
            wy = jnp.maximum(jnp.minimum(y2i, y2j) - jnp.maximum(y1i, y1j), 0.0)
            inter = wx * wy
            union = area_i + area_j - inter
            iou = inter / union
            ew = jnp.maximum(jnp.maximum(x2i, x2j) - jnp.minimum(x1i, x1j), 0.0)
            eh = jnp.maximum(jnp.maximum(y2i, y2j) - jnp.minimum(y1i, y1j), 0.0)
            area_e = ew * eh
            giou = iou - (area_e - union) / area_e
            m = jnp.where(giou > T_B, 1.0, 0.0)
            ab_s[r0:r0 + _CHUNK, pl.ds(c0, _CHUNK)] = m.astype(jnp.bfloat16)

            @pl.when(cj > ci)
            def _():
                mt = jnp.swapaxes(m, 0, 1)
                ab_s[pl.ds(c0, _CHUNK), r0:r0 + _CHUNK] = mt.astype(jnp.bfloat16)

            return s + jnp.sum(m) * jnp.where(cj > ci, 2.0, 1.0)

        s0 = jax.lax.fori_loop(ci, _NCH, giou_tile, s0)

    def sweep_cond(st):
        t, _, changed = st
        return jnp.logical_and(changed, t < _N)

    def sweep(st):
        t, prev, _ = st

        def chunk(k, s):
            r0 = k * _CHUNK
            lhs = ab_s[pl.ds(r0, _CHUNK), :]
            cnt = jax.lax.dot_general(
                lhs, ab_s[...], (((1,), (0,)), ((), ())),
                preferred_element_type=jnp.float32)
            new = jnp.where(lhs.astype(jnp.float32) + cnt > _EPS_ADJ, 1.0, 0.0)
            ab_s[pl.ds(r0, _CHUNK), :] = new.astype(jnp.bfloat16)
            return s + jnp.sum(new)

        ns = jax.lax.fori_loop(0, _NCH, chunk, jnp.float32(0.0))
        return (t + 1, ns, ns > prev)

    _, _, _ = jax.lax.while_loop(
        sweep_cond, sweep, (jnp.int32(0), s0, jnp.bool_(True)))

    bb16 = bb8_ref[0].astype(jnp.bfloat16)
    m8 = jax.lax.dot_general(
        ab_s[...], bb16, (((1,), (0,)), ((), ())),
        preferred_element_type=jnp.float32)
    adj_ref[0] = ab_s[0:_N, 0:_N].astype(jnp.float32)
    agg_ref[0] = m8[0:_N, 0:4] / (m8[0:_N, 4:5] + _EPS_DEN)


def kernel(bboxes, logits):
    B, n, _ = bboxes.shape
    f32 = jnp.float32
    bbp = jnp.pad(bboxes.astype(f32), ((0, 0), (0, _NP - n), (0, 0)))
    bb8 = jnp.concatenate(
        [bbp, jnp.ones((B, _NP, 1), f32), jnp.zeros((B, _NP, 3), f32)], axis=-1)
    crows = jnp.pad(jnp.swapaxes(bbp, 1, 2), ((0, 0), (0, 4), (0, 0)))

    adj, agg = pl.pallas_call(
        _body,
        grid=(B,),
        in_specs=[
            pl.BlockSpec((1, _NP, 8), lambda b: (b, 0, 0)),
            pl.BlockSpec((1, 8, _NP), lambda b: (b, 0, 0)),
        ],
        out_specs=[
            pl.BlockSpec((1, n, n), lambda b: (b, 0, 0)),
            pl.BlockSpec((1, n, 4), lambda b: (b, 0, 0)),
        ],
        out_shape=[
            jax.ShapeDtypeStruct((B, n, n), f32),
            jax.ShapeDtypeStruct((B, n, 4), f32),
        ],
        scratch_shapes=[pltpu.VMEM((_NP, _NP), jnp.bfloat16)],
        compiler_params=pltpu.CompilerParams(
            dimension_semantics=("parallel",),
            vmem_limit_bytes=96 * 1024 * 1024,
        ),
    )(bb8, crows)
    return (agg, logits, adj)


# speedup vs baseline: 1.3631x; 1.3631x over previous
import jax
import jax.numpy as jnp
from jax.experimental import pallas as pl
from jax.experimental.pallas import tpu as pltpu

T_B = 0.9
_EPS_ADJ = 1e-6
_EPS_DEN = 1e-6
_N = 900
_NP = 1024
_CHUNK = 128
_NCH = _NP // _CHUNK


def _body(bb8_ref, crows_ref, adj_ref, agg_ref, ab_s):
    crows = crows_ref[0]
    x1j = crows[0:1, :] - 0.5 * crows[2:3, :]
    y1j = crows[1:2, :] - 0.5 * crows[3:4, :]
    x2j = crows[0:1, :] + 0.5 * crows[2:3, :]
    y2j = crows[1:2, :] + 0.5 * crows[3:4, :]
    area_j = (x2j - x1j) * (y2j - y1j)

    def giou_chunk(k, s):
        r0 = k * _CHUNK
        c = bb8_ref[0, pl.ds(r0, _CHUNK), :]
        x1i = c[:, 0:1] - 0.5 * c[:, 2:3]
        y1i = c[:, 1:2] - 0.5 * c[:, 3:4]
        x2i = c[:, 0:1] + 0.5 * c[:, 2:3]
        y2i = c[:, 1:2] + 0.5 * c[:, 3:4]
        area_i = (x2i - x1i) * (y2i - y1i)
        wx = jnp.maximum(jnp.minimum(x2i, x2j) - jnp.maximum(x1i, x1j), 0.0)
        wy = jnp.maximum(jnp.minimum(y2i, y2j) - jnp.maximum(y1i, y1j), 0.0)
        inter = wx * wy
        union = area_i + area_j - inter
        iou = inter / union
        ew = jnp.maximum(jnp.maximum(x2i, x2j) - jnp.minimum(x1i, x1j), 0.0)
        eh = jnp.maximum(jnp.maximum(y2i, y2j) - jnp.minimum(y1i, y1j), 0.0)
        area_e = ew * eh
        giou = iou - (area_e - union) / area_e
        m = jnp.where(giou > T_B, 1.0, 0.0)
        ab_s[pl.ds(r0, _CHUNK), :] = m.astype(jnp.bfloat16)
        return s + jnp.sum(m)

    s0 = jax.lax.fori_loop(0, _NCH, giou_chunk, jnp.float32(0.0))

    def sweep_cond(st):
        t, _, changed = st
        return jnp.logical_and(changed, t < _N)

    def sweep(st):
        t, prev, _ = st

        def chunk(k, s):
            r0 = k * _CHUNK
            lhs = ab_s[pl.ds(r0, _CHUNK), :]
            cnt = jax.lax.dot_general(
                lhs, ab_s[...], (((1,), (0,)), ((), ())),
                preferred_element_type=jnp.float32)
            new = jnp.where(lhs.astype(jnp.float32) + cnt > _EPS_ADJ, 1.0, 0.0)
            ab_s[pl.ds(r0, _CHUNK), :] = new.astype(jnp.bfloat16)
            return s + jnp.sum(new)

        ns = jax.lax.fori_loop(0, _NCH, chunk, jnp.float32(0.0))
        return (t + 1, ns, ns > prev)

    _, _, _ = jax.lax.while_loop(
        sweep_cond, sweep, (jnp.int32(0), s0, jnp.bool_(True)))

    bb16 = bb8_ref[0].astype(jnp.bfloat16)
    m8 = jax.lax.dot_general(
        ab_s[...], bb16, (((1,), (0,)), ((), ())),
        preferred_element_type=jnp.float32)
    adj_ref[0] = ab_s[0:_N, 0:_N].astype(jnp.float32)
    agg_ref[0] = m8[0:_N, 0:4] / (m8[0:_N, 4:5] + _EPS_DEN)


def kernel(bboxes, logits):
    B, n, _ = bboxes.shape
    f32 = jnp.float32
    bbp = jnp.pad(bboxes.astype(f32), ((0, 0), (0, _NP - n), (0, 0)))
    bb8 = jnp.concatenate(
        [bbp, jnp.ones((B, _NP, 1), f32), jnp.zeros((B, _NP, 3), f32)], axis=-1)
    crows = jnp.pad(jnp.swapaxes(bbp, 1, 2), ((0, 0), (0, 4), (0, 0)))

    adj, agg = pl.pallas_call(
        _body,
        grid=(B,),
        in_specs=[
            pl.BlockSpec((1, _NP, 8), lambda b: (b, 0, 0)),
            pl.BlockSpec((1, 8, _NP), lambda b: (b, 0, 0)),
        ],
        out_specs=[
            pl.BlockSpec((1, n, n), lambda b: (b, 0, 0)),
            pl.BlockSpec((1, n, 4), lambda b: (b, 0, 0)),
        ],
        out_shape=[
            jax.ShapeDtypeStruct((B, n, n), f32),
            jax.ShapeDtypeStruct((B, n, 4), f32),
        ],
        scratch_shapes=[pltpu.VMEM((_NP, _NP), jnp.bfloat16)],
        compiler_params=pltpu.CompilerParams(
            dimension_semantics=("parallel",),
            vmem_limit_bytes=96 * 1024 * 1024,
        ),
    )(bb8, crows)
    return (agg, logits, adj)


# speedup vs baseline: 1.4690x; 1.0777x over previous
import jax
import jax.numpy as jnp
from jax.experimental import pallas as pl
from jax.experimental.pallas import tpu as pltpu

T_B = 0.9
_EPS_ADJ = 1e-6
_EPS_DEN = 1e-6
_N = 900
_NP = 1024
_CHUNK = 128
_NCH = _NP // _CHUNK
_G = 2


def _body(bb8_ref, crows_ref, adj_ref, agg_ref, ab_s):
    jvecs = []
    for g in range(_G):
        crows = crows_ref[g]
        x1j = crows[0:1, :] - 0.5 * crows[2:3, :]
        y1j = crows[1:2, :] - 0.5 * crows[3:4, :]
        x2j = crows[0:1, :] + 0.5 * crows[2:3, :]
        y2j = crows[1:2, :] + 0.5 * crows[3:4, :]
        area_j = (x2j - x1j) * (y2j - y1j)
        jvecs.append((x1j, y1j, x2j, y2j, area_j))

    def giou_chunk(k, carry):
        r0 = k * _CHUNK
        sums = []
        for g in range(_G):
            x1j, y1j, x2j, y2j, area_j = jvecs[g]
            c = bb8_ref[g, pl.ds(r0, _CHUNK), :]
            x1i = c[:, 0:1] - 0.5 * c[:, 2:3]
            y1i = c[:, 1:2] - 0.5 * c[:, 3:4]
            x2i = c[:, 0:1] + 0.5 * c[:, 2:3]
            y2i = c[:, 1:2] + 0.5 * c[:, 3:4]
            area_i = (x2i - x1i) * (y2i - y1i)
            wx = jnp.maximum(jnp.minimum(x2i, x2j) - jnp.maximum(x1i, x1j), 0.0)
            wy = jnp.maximum(jnp.minimum(y2i, y2j) - jnp.maximum(y1i, y1j), 0.0)
            inter = wx * wy
            union = area_i + area_j - inter
            iou = inter / union
            ew = jnp.maximum(jnp.maximum(x2i, x2j) - jnp.minimum(x1i, x1j), 0.0)
            eh = jnp.maximum(jnp.maximum(y2i, y2j) - jnp.minimum(y1i, y1j), 0.0)
            area_e = ew * eh
            giou = iou - (area_e - union) / area_e
            m = jnp.where(giou > T_B, 1.0, 0.0)
            ab_s[g, pl.ds(r0, _CHUNK), :] = m.astype(jnp.bfloat16)
            sums.append(jnp.sum(m))
        return (carry[0] + sums[0], carry[1] + sums[1])

    s0a, s0b = jax.lax.fori_loop(
        0, _NCH, giou_chunk, (jnp.float32(0.0), jnp.float32(0.0)))

    def sweep_cond(st):
        t, _, _, ca, cb = st
        return jnp.logical_and(jnp.logical_or(ca, cb), t < _N)

    def sweep(st):
        t, pa, pb, _, _ = st

        def chunk(k, carry):
            r0 = k * _CHUNK
            sums = []
            for g in range(_G):
                lhs = ab_s[g, pl.ds(r0, _CHUNK), :]
                cnt = jax.lax.dot_general(
                    lhs, ab_s[g], (((1,), (0,)), ((), ())),
                    preferred_element_type=jnp.float32)
                new = jnp.where(
                    lhs.astype(jnp.float32) + cnt > _EPS_ADJ, 1.0, 0.0)
                ab_s[g, pl.ds(r0, _CHUNK), :] = new.astype(jnp.bfloat16)
                sums.append(jnp.sum(new))
            return (carry[0] + sums[0], carry[1] + sums[1])

        nsa, nsb = jax.lax.fori_loop(
            0, _NCH, chunk, (jnp.float32(0.0), jnp.float32(0.0)))
        return (t + 1, nsa, nsb, nsa > pa, nsb > pb)

    jax.lax.while_loop(
        sweep_cond, sweep,
        (jnp.int32(0), s0a, s0b, jnp.bool_(True), jnp.bool_(True)))

    for g in range(_G):
        bb16 = bb8_ref[g].astype(jnp.bfloat16)
        m8 = jax.lax.dot_general(
            ab_s[g], bb16, (((1,), (0,)), ((), ())),
            preferred_element_type=jnp.float32)
        adj_ref[g] = ab_s[g, 0:_N, 0:_N].astype(jnp.float32)
        agg_ref[g] = m8[0:_N, 0:4] / (m8[0:_N, 4:5] + _EPS_DEN)


def kernel(bboxes, logits):
    B, n, _ = bboxes.shape
    f32 = jnp.float32
    bbp = jnp.pad(bboxes.astype(f32), ((0, 0), (0, _NP - n), (0, 0)))
    bb8 = jnp.concatenate(
        [bbp, jnp.ones((B, _NP, 1), f32), jnp.zeros((B, _NP, 3), f32)], axis=-1)
    crows = jnp.pad(jnp.swapaxes(bbp, 1, 2), ((0, 0), (0, 4), (0, 0)))

    adj, agg = pl.pallas_call(
        _body,
        grid=(B // _G,),
        in_specs=[
            pl.BlockSpec((_G, _NP, 8), lambda b: (b, 0, 0)),
            pl.BlockSpec((_G, 8, _NP), lambda b: (b, 0, 0)),
        ],
        out_specs=[
            pl.BlockSpec((_G, n, n), lambda b: (b, 0, 0)),
            pl.BlockSpec((_G, n, 4), lambda b: (b, 0, 0)),
        ],
        out_shape=[
            jax.ShapeDtypeStruct((B, n, n), f32),
            jax.ShapeDtypeStruct((B, n, 4), f32),
        ],
        scratch_shapes=[pltpu.VMEM((_G, _NP, _NP), jnp.bfloat16)],
        compiler_params=pltpu.CompilerParams(
            dimension_semantics=("parallel",),
            vmem_limit_bytes=100 * 1024 * 1024,
        ),
    )(bb8, crows)
    return (agg, logits, adj)
